# baseline (speedup 1.0000x reference)
import jax
import jax.numpy as jnp
from jax import lax
from jax.experimental import pallas as pl
from jax.experimental.pallas import tpu as pltpu
from jax.experimental.pallas import tpu_sc as plsc

N = 10000
NP = 10240
E = 160000
EP = 163840
D = 256
DH = 128
NT = 16
CHUNK = 128
NCH = EP // NT // CHUNK
NCH_DEG = EP // (2 * NT) // CHUNK
RPT = NP // NT
SEG = 40
NSEG = NCH // SEG

_sc_mesh = plsc.VectorSubcoreMesh(core_axis_name="c", subcore_axis_name="s")


def _deg_body(col_hbm, deg_hbm, idx_col, ones_v, zb, acc):
    cid = lax.axis_index("c")
    sid = lax.axis_index("s")
    wid = cid * NT + sid
    pltpu.sync_copy(col_hbm.at[wid], idx_col)

    def fill(r, _):
        ones_v[r, pl.ds(0, 16)] = jnp.ones((16,), jnp.float32)
        zb[r, pl.ds(0, 16)] = jnp.zeros((16,), jnp.float32)
        return 0

    lax.fori_loop(0, CHUNK, fill, 0)
    for k in range(RPT // CHUNK):
        pltpu.sync_copy(zb, acc.at[pl.ds(sid * RPT + k * CHUNK, CHUNK)])
    plsc.subcore_barrier()

    def body(g, _):
        pltpu.sync_copy(ones_v, acc.at[idx_col.at[g]], add=True)
        return 0

    lax.fori_loop(0, NCH_DEG, body, 0)
    plsc.subcore_barrier()
    pltpu.sync_copy(acc.at[pl.ds(sid * RPT, RPT)],
                    deg_hbm.at[pl.ds(cid * NP + sid * RPT, RPT)])


_deg_call = pl.kernel(
    _deg_body,
    out_type=jax.ShapeDtypeStruct((2 * NP, 16), jnp.float32),
    mesh=_sc_mesh,
    scratch_types=[
        pltpu.VMEM((NCH_DEG, CHUNK), jnp.int32),
        pltpu.VMEM((CHUNK, 16), jnp.float32),
        pltpu.VMEM((CHUNK, 16), jnp.float32),
        pltpu.VMEM_SHARED((NP, 16), jnp.float32),
    ],
)


def _segsum_body(y_hbm, row_hbm, col_hbm, out_hbm,
                 idx_row, idx_col, b0, b1, acc, g0, g1, s0, s1):
    cid = lax.axis_index("c")
    sid = lax.axis_index("s")
    wid = cid * NT + sid

    def fill(r, _):
        for j in range(DH // 16):
            b0[r, pl.ds(j * 16, 16)] = jnp.zeros((16,), jnp.float32)
        return 0

    lax.fori_loop(0, CHUNK, fill, 0)
    for k in range(RPT // CHUNK):
        pltpu.sync_copy(b0, acc.at[pl.ds(sid * RPT + k * CHUNK, CHUNK)])
    plsc.subcore_barrier()

    def gath(c, buf, gsem):
        return pltpu.async_copy(y_hbm.at[idx_row.at[c]], buf, gsem)

    def scat(c, buf, ssem):
        return pltpu.async_copy(buf, acc.at[idx_col.at[c]], ssem, add=True)

    def seg(si, _):
        pltpu.sync_copy(row_hbm.at[wid, pl.ds(si * SEG, SEG)], idx_row)
        pltpu.sync_copy(col_hbm.at[sid, pl.ds(si * SEG, SEG)], idx_col)

        gath(0, b0, g0).wait()

        def pair(k, _):
            c = 2 * k
            sc0 = scat(c, b0, s0)
            cp1 = gath(c + 1, b1, g1)
            cp1.wait()
            sc0.wait()
            sc1 = scat(c + 1, b1, s1)
            cp0 = gath(c + 2, b0, g0)
            cp0.wait()
            sc1.wait()
            return 0

        lax.fori_loop(0, SEG // 2 - 1, pair, 0)
        c = SEG - 2
        sc0 = scat(c, b0, s0)
        cp1 = gath(c + 1, b1, g1)
        cp1.wait()
        sc0.wait()
        sc1 = scat(c + 1, b1, s1)
        sc1.wait()
        return 0

    lax.fori_loop(0, NSEG, seg, 0)
    plsc.subcore_barrier()
    pltpu.sync_copy(acc.at[pl.ds(sid * RPT, RPT)],
                    out_hbm.at[pl.ds(cid * NP + sid * RPT, RPT)])


_segsum_call = pl.kernel(
    _segsum_body,
    out_type=jax.ShapeDtypeStruct((2 * NP, DH), jnp.float32),
    mesh=_sc_mesh,
    scratch_types=(
        [pltpu.VMEM((SEG, CHUNK), jnp.int32)] * 2
        + [pltpu.VMEM((CHUNK, DH), jnp.float32)] * 2
        + [pltpu.VMEM_SHARED((NP, DH), jnp.float32)]
        + [pltpu.SemaphoreType.DMA] * 4
    ),
)


R = 2000
_NB = N // R


def _scales(d0, d1):
    deg = d0[:, :1] + d1[:, :1] + 1.0
    dis = lax.rsqrt(deg)
    return dis, 1.0 / deg


def _mmA_body(d0, d1, x, w, b, y_out, a_out):
    dis, inv = _scales(d0[...], d1[...])
    xw = jnp.dot(x[...], w[...], preferred_element_type=jnp.float32)
    y_out[0] = dis * xw[:, :DH]
    y_out[1] = dis * xw[:, DH:]
    a_out[...] = inv * xw + b[...]


def _mmB_body(d0, d1, s, a1, w, b, y_out, a_out):
    dis, inv = _scales(d0[...], d1[...])
    h = dis * jnp.concatenate([s[0], s[1]], axis=1) + a1[...]
    h = jnp.maximum(h, 0.0)
    xw = jnp.dot(h, w[...], preferred_element_type=jnp.float32)
    y_out[0] = dis * xw[:, :DH]
    y_out[1] = dis * xw[:, DH:]
    a_out[...] = inv * xw + b[...]


def _mmC_body(d0, d1, s, a2, out):
    dis, _ = _scales(d0[...], d1[...])
    out[...] = dis * jnp.concatenate([s[0], s[1]], axis=1) + a2[...]


_deg_spec = pl.BlockSpec((R, 16), lambda i: (i, 0))
_row_spec = pl.BlockSpec((R, D), lambda i: (i, 0))
_half_spec = pl.BlockSpec((2, R, DH), lambda i: (0, i, 0))
_w_spec = pl.BlockSpec((D, D), lambda i: (0, 0))
_b_spec = pl.BlockSpec((D,), lambda i: (0,))

_mmA_call = pl.pallas_call(
    _mmA_body,
    grid=(_NB,),
    in_specs=[_deg_spec, _deg_spec, _row_spec, _w_spec, _b_spec],
    out_specs=[_half_spec, _row_spec],
    out_shape=[jax.ShapeDtypeStruct((2, NP, DH), jnp.float32),
               jax.ShapeDtypeStruct((N, D), jnp.float32)],
)

_mmB_call = pl.pallas_call(
    _mmB_body,
    grid=(_NB,),
    in_specs=[_deg_spec, _deg_spec, _half_spec, _row_spec, _w_spec, _b_spec],
    out_specs=[_half_spec, _row_spec],
    out_shape=[jax.ShapeDtypeStruct((2, NP, DH), jnp.float32),
               jax.ShapeDtypeStruct((N, D), jnp.float32)],
)

_mmC_call = pl.pallas_call(
    _mmC_body,
    grid=(_NB,),
    in_specs=[_deg_spec, _deg_spec, _half_spec, _row_spec],
    out_specs=_row_spec,
    out_shape=jax.ShapeDtypeStruct((N, D), jnp.float32),
)


@jax.jit
def kernel(x, edge_index, W1, b1, W2, b2):
    row = edge_index[0].astype(jnp.int32)
    col = edge_index[1].astype(jnp.int32)
    pad = EP - E
    row_p = jnp.concatenate([row, jnp.zeros((pad,), jnp.int32)])
    col_p = jnp.concatenate([col, jnp.full((pad,), N, jnp.int32)])
    col_deg = col_p.reshape(2 * NT, NCH_DEG, CHUNK)
    col_seg = col_p.reshape(NT, NCH, CHUNK)
    row2 = jnp.stack([row_p, row_p + NP]).reshape(2 * NT, NCH, CHUNK)

    deg2 = _deg_call(col_deg)
    d0 = deg2[:N]
    d1 = deg2[NP:NP + N]

    y1, a1 = _mmA_call(d0, d1, x, W1, b1)
    s1 = _segsum_call(y1.reshape(2 * NP, DH), row2, col_seg)
    y2, a2 = _mmB_call(d0, d1, s1.reshape(2, NP, DH), a1, W2, b2)
    s2 = _segsum_call(y2.reshape(2 * NP, DH), row2, col_seg)
    return _mmC_call(d0, d1, s2.reshape(2, NP, DH), a2)


# speedup vs baseline: 1.8041x; 1.8041x over previous
import jax
import jax.numpy as jnp
from jax import lax
from jax.experimental import pallas as pl
from jax.experimental.pallas import tpu as pltpu
from jax.experimental.pallas import tpu_sc as plsc

N = 10000
NP = 10240
E = 160000
EP = 163840
D = 256
DH = 128
NT = 16
CHUNK = 128
NCH = EP // NT // CHUNK
NCH_DEG = EP // (2 * NT) // CHUNK
RPT = NP // NT
SEG = 40
NSEG = NCH // SEG

_sc_mesh = plsc.VectorSubcoreMesh(core_axis_name="c", subcore_axis_name="s")


def _deg_body(col_hbm, deg_hbm, idx_col, ones_v, zb, acc):
    cid = lax.axis_index("c")
    sid = lax.axis_index("s")
    wid = cid * NT + sid
    pltpu.sync_copy(col_hbm.at[wid], idx_col)

    def fill(r, _):
        ones_v[r, pl.ds(0, 16)] = jnp.ones((16,), jnp.float32)
        zb[r, pl.ds(0, 16)] = jnp.zeros((16,), jnp.float32)
        return 0

    lax.fori_loop(0, CHUNK, fill, 0)
    for k in range(RPT // CHUNK):
        pltpu.sync_copy(zb, acc.at[pl.ds(sid * RPT + k * CHUNK, CHUNK)])
    plsc.subcore_barrier()

    def body(g, _):
        pltpu.sync_copy(ones_v, acc.at[idx_col.at[g]], add=True)
        return 0

    lax.fori_loop(0, NCH_DEG, body, 0)
    plsc.subcore_barrier()
    pltpu.sync_copy(acc.at[pl.ds(sid * RPT, RPT)],
                    deg_hbm.at[pl.ds(cid * NP + sid * RPT, RPT)])


_deg_call = pl.kernel(
    _deg_body,
    out_type=jax.ShapeDtypeStruct((2 * NP, 16), jnp.float32),
    mesh=_sc_mesh,
    scratch_types=[
        pltpu.VMEM((NCH_DEG, CHUNK), jnp.int32),
        pltpu.VMEM((CHUNK, 16), jnp.float32),
        pltpu.VMEM((CHUNK, 16), jnp.float32),
        pltpu.VMEM_SHARED((NP, 16), jnp.float32),
    ],
)


def _segsum_body(y_hbm, row_hbm, col_hbm, out_hbm,
                 idx_row, idx_col, b0, b1, acc, g0, g1, s0, s1):
    cid = lax.axis_index("c")
    sid = lax.axis_index("s")
    wid = cid * NT + sid

    def fill(r, _):
        for j in range(DH // 16):
            b0[r, pl.ds(j * 16, 16)] = jnp.zeros((16,), jnp.float32)
        return 0

    lax.fori_loop(0, CHUNK, fill, 0)
    for k in range(RPT // CHUNK):
        pltpu.sync_copy(b0, acc.at[pl.ds(sid * RPT + k * CHUNK, CHUNK)])
    plsc.subcore_barrier()

    def gath(c, buf, gsem):
        return pltpu.async_copy(y_hbm.at[idx_row.at[c]], buf, gsem)

    def scat(c, buf, ssem):
        return pltpu.async_copy(buf, acc.at[idx_col.at[c]], ssem, add=True)

    def seg(si, _):
        pltpu.sync_copy(row_hbm.at[wid, pl.ds(si * SEG, SEG)], idx_row)
        pltpu.sync_copy(col_hbm.at[sid, pl.ds(si * SEG, SEG)], idx_col)

        gath(0, b0, g0).wait()

        def pair(k, _):
            c = 2 * k
            sc0 = scat(c, b0, s0)
            cp1 = gath(c + 1, b1, g1)
            cp1.wait()
            sc0.wait()
            sc1 = scat(c + 1, b1, s1)
            cp0 = gath(c + 2, b0, g0)
            cp0.wait()
            sc1.wait()
            return 0

        lax.fori_loop(0, SEG // 2 - 1, pair, 0)
        c = SEG - 2
        sc0 = scat(c, b0, s0)
        cp1 = gath(c + 1, b1, g1)
        cp1.wait()
        sc0.wait()
        sc1 = scat(c + 1, b1, s1)
        sc1.wait()
        return 0

    lax.fori_loop(0, NSEG, seg, 0)
    plsc.subcore_barrier()
    pltpu.sync_copy(acc.at[pl.ds(sid * RPT, RPT)],
                    out_hbm.at[pl.ds(cid * NP + sid * RPT, RPT)])


_segsum_call = pl.kernel(
    _segsum_body,
    out_type=jax.ShapeDtypeStruct((2 * NP, DH), jnp.float32),
    mesh=_sc_mesh,
    scratch_types=(
        [pltpu.VMEM((SEG, CHUNK), jnp.int32)] * 2
        + [pltpu.VMEM((CHUNK, DH), jnp.float32)] * 2
        + [pltpu.VMEM_SHARED((NP, DH), jnp.float32)]
        + [pltpu.SemaphoreType.DMA] * 4
    ),
)


R = 2000
_NB = N // R


def _scales(d0, d1):
    deg = d0[:, :1] + d1[:, :1] + 1.0
    dis = lax.rsqrt(deg)
    return dis, 1.0 / deg


def _mmA_body(d0, d1, x, w, b, y_out, a_out):
    dis, inv = _scales(d0[...], d1[...])
    xw = jnp.dot(x[...], w[...], preferred_element_type=jnp.float32)
    y_out[0] = dis * xw[:, :DH]
    y_out[1] = dis * xw[:, DH:]
    a_out[...] = inv * xw + b[...]


def _mmB_body(d0, d1, s, a1, w, b, y_out, a_out):
    dis, inv = _scales(d0[...], d1[...])
    h = dis * jnp.concatenate([s[0], s[1]], axis=1) + a1[...]
    h = jnp.maximum(h, 0.0)
    xw = jnp.dot(h, w[...], preferred_element_type=jnp.float32)
    y_out[0] = dis * xw[:, :DH]
    y_out[1] = dis * xw[:, DH:]
    a_out[...] = inv * xw + b[...]


def _mmC_body(d0, d1, s, a2, out):
    dis, _ = _scales(d0[...], d1[...])
    out[...] = dis * jnp.concatenate([s[0], s[1]], axis=1) + a2[...]


_deg_spec = pl.BlockSpec((R, 16), lambda i: (i, 0))
_row_spec = pl.BlockSpec((R, D), lambda i: (i, 0))
_half_spec = pl.BlockSpec((2, R, DH), lambda i: (0, i, 0))
_w_spec = pl.BlockSpec((D, D), lambda i: (0, 0))
_b_spec = pl.BlockSpec((D,), lambda i: (0,))

_mmA_call = pl.pallas_call(
    _mmA_body,
    grid=(_NB,),
    in_specs=[_deg_spec, _deg_spec, _row_spec, _w_spec, _b_spec],
    out_specs=[_half_spec, _row_spec],
    out_shape=[jax.ShapeDtypeStruct((2, NP, DH), jnp.float32),
               jax.ShapeDtypeStruct((N, D), jnp.float32)],
)

_mmB_call = pl.pallas_call(
    _mmB_body,
    grid=(_NB,),
    in_specs=[_deg_spec, _deg_spec, _half_spec, _row_spec, _w_spec, _b_spec],
    out_specs=[_half_spec, _row_spec],
    out_shape=[jax.ShapeDtypeStruct((2, NP, DH), jnp.float32),
               jax.ShapeDtypeStruct((N, D), jnp.float32)],
)

_mmC_call = pl.pallas_call(
    _mmC_body,
    grid=(_NB,),
    in_specs=[_deg_spec, _deg_spec, _half_spec, _row_spec],
    out_specs=_row_spec,
    out_shape=jax.ShapeDtypeStruct((N, D), jnp.float32),
)


@jax.jit
def kernel(x, edge_index, W1, b1, W2, b2):
    row = edge_index[0].astype(jnp.int32)
    col = edge_index[1].astype(jnp.int32)
    pad = EP - E
    pr = jnp.arange(pad, dtype=jnp.int32)
    row_p = jnp.concatenate([row, pr % N])
    col_p = jnp.concatenate([col, N + pr % (NP - N)])
    col_deg = col_p.reshape(2 * NT, NCH_DEG, CHUNK)
    col_seg = col_p.reshape(NT, NCH, CHUNK)
    row2 = jnp.stack([row_p, row_p + NP]).reshape(2 * NT, NCH, CHUNK)

    deg2 = _deg_call(col_deg)
    d0 = deg2[:N]
    d1 = deg2[NP:NP + N]

    y1, a1 = _mmA_call(d0, d1, x, W1, b1)
    s1 = _segsum_call(y1.reshape(2 * NP, DH), row2, col_seg)
    y2, a2 = _mmB_call(d0, d1, s1.reshape(2, NP, DH), a1, W2, b2)
    s2 = _segsum_call(y2.reshape(2 * NP, DH), row2, col_seg)
    return _mmC_call(d0, d1, s2.reshape(2, NP, DH), a2)


# speedup vs baseline: 1.8519x; 1.0265x over previous
import jax
import jax.numpy as jnp
from jax import lax
from jax.experimental import pallas as pl
from jax.experimental.pallas import tpu as pltpu
from jax.experimental.pallas import tpu_sc as plsc

N = 10000
NP = 10240
E = 160000
EP = 163840
D = 256
DH = 128
NT = 16
CHUNK = 128
NCH = EP // NT // CHUNK
NCH_DEG = EP // (2 * NT) // CHUNK
RPT = NP // NT
SEG = 40
NSEG = NCH // SEG

_sc_mesh = plsc.VectorSubcoreMesh(core_axis_name="c", subcore_axis_name="s")


def _deg_body(col_hbm, deg_hbm, idx_col, ones_v, zb, acc):
    cid = lax.axis_index("c")
    sid = lax.axis_index("s")
    wid = cid * NT + sid
    pltpu.sync_copy(col_hbm.at[wid], idx_col)

    def fill(r, _):
        ones_v[r, pl.ds(0, 16)] = jnp.ones((16,), jnp.float32)
        zb[r, pl.ds(0, 16)] = jnp.zeros((16,), jnp.float32)
        return 0

    lax.fori_loop(0, CHUNK, fill, 0)
    for k in range(RPT // CHUNK):
        pltpu.sync_copy(zb, acc.at[pl.ds(sid * RPT + k * CHUNK, CHUNK)])
    plsc.subcore_barrier()

    def body(g, _):
        pltpu.sync_copy(ones_v, acc.at[idx_col.at[g]], add=True)
        return 0

    lax.fori_loop(0, NCH_DEG, body, 0)
    plsc.subcore_barrier()
    pltpu.sync_copy(acc.at[pl.ds(sid * RPT, RPT)],
                    deg_hbm.at[pl.ds(cid * NP + sid * RPT, RPT)])


_deg_call = pl.kernel(
    _deg_body,
    out_type=jax.ShapeDtypeStruct((2 * NP, 16), jnp.float32),
    mesh=_sc_mesh,
    scratch_types=[
        pltpu.VMEM((NCH_DEG, CHUNK), jnp.int32),
        pltpu.VMEM((CHUNK, 16), jnp.float32),
        pltpu.VMEM((CHUNK, 16), jnp.float32),
        pltpu.VMEM_SHARED((NP, 16), jnp.float32),
    ],
)


def _segsum_body(y_hbm, row_hbm, col_hbm, out_hbm,
                 idx_row, idx_col, b0, b1, acc, g0, g1, s0, s1):
    cid = lax.axis_index("c")
    sid = lax.axis_index("s")
    wid = cid * NT + sid

    def fill(r, _):
        for j in range(DH // 16):
            b0[r, pl.ds(j * 16, 16)] = jnp.zeros((16,), jnp.float32)
        return 0

    lax.fori_loop(0, CHUNK, fill, 0)
    zcp = [pltpu.async_copy(b0, acc.at[pl.ds(sid * RPT + k * CHUNK, CHUNK)],
                            g0)
           for k in range(RPT // CHUNK)]
    for cp in zcp:
        cp.wait()
    plsc.subcore_barrier()

    def gath(c, buf, gsem):
        return pltpu.async_copy(y_hbm.at[idx_row.at[c]], buf, gsem)

    def scat(c, buf, ssem):
        return pltpu.async_copy(buf, acc.at[idx_col.at[c]], ssem, add=True)

    def seg(si, _):
        pltpu.sync_copy(row_hbm.at[wid, pl.ds(si * SEG, SEG)], idx_row)
        pltpu.sync_copy(col_hbm.at[sid, pl.ds(si * SEG, SEG)], idx_col)

        gath(0, b0, g0).wait()

        def pair(k, _):
            c = 2 * k
            sc0 = scat(c, b0, s0)
            cp1 = gath(c + 1, b1, g1)
            cp1.wait()
            sc0.wait()
            sc1 = scat(c + 1, b1, s1)
            cp0 = gath(c + 2, b0, g0)
            cp0.wait()
            sc1.wait()
            return 0

        lax.fori_loop(0, SEG // 2 - 1, pair, 0)
        c = SEG - 2
        sc0 = scat(c, b0, s0)
        cp1 = gath(c + 1, b1, g1)
        cp1.wait()
        sc0.wait()
        sc1 = scat(c + 1, b1, s1)
        sc1.wait()
        return 0

    lax.fori_loop(0, NSEG, seg, 0)
    plsc.subcore_barrier()
    pltpu.sync_copy(acc.at[pl.ds(sid * RPT, RPT)],
                    out_hbm.at[pl.ds(cid * NP + sid * RPT, RPT)])


_segsum_call = pl.kernel(
    _segsum_body,
    out_type=jax.ShapeDtypeStruct((2 * NP, DH), jnp.float32),
    mesh=_sc_mesh,
    scratch_types=(
        [pltpu.VMEM((SEG, CHUNK), jnp.int32)] * 2
        + [pltpu.VMEM((CHUNK, DH), jnp.float32)] * 2
        + [pltpu.VMEM_SHARED((NP, DH), jnp.float32)]
        + [pltpu.SemaphoreType.DMA] * 4
    ),
)


R = 2048
_NB = NP // R


def _scales(d0, d1):
    deg = d0[:, :1] + d1[:, :1] + 1.0
    dis = lax.rsqrt(deg)
    return dis, 1.0 / deg


def _mmA_body(d0, d1, x, w, b, y_out, a_out):
    dis, inv = _scales(d0[...], d1[...])
    xw = jnp.dot(x[...], w[...], preferred_element_type=jnp.float32)
    y_out[0] = dis * xw[:, :DH]
    y_out[1] = dis * xw[:, DH:]
    a_out[...] = inv * xw + b[...]


def _mmB_body(d0, d1, s, a1, w, b, y_out, a_out):
    dis, inv = _scales(d0[...], d1[...])
    h = dis * jnp.concatenate([s[0], s[1]], axis=1) + a1[...]
    h = jnp.maximum(h, 0.0)
    xw = jnp.dot(h, w[...], preferred_element_type=jnp.float32)
    y_out[0] = dis * xw[:, :DH]
    y_out[1] = dis * xw[:, DH:]
    a_out[...] = inv * xw + b[...]


def _mmC_body(d0, d1, s, a2, out):
    dis, _ = _scales(d0[...], d1[...])
    out[...] = dis * jnp.concatenate([s[0], s[1]], axis=1) + a2[...]


_deg_spec = pl.BlockSpec((R, 16), lambda i: (i, 0))
_deg1_spec = pl.BlockSpec((R, 16), lambda i: (i + _NB, 0))
_row_spec = pl.BlockSpec((R, D), lambda i: (i, 0))
_half_spec = pl.BlockSpec((2, R, DH), lambda i: (0, i, 0))
_w_spec = pl.BlockSpec((D, D), lambda i: (0, 0))
_b_spec = pl.BlockSpec((D,), lambda i: (0,))

_mmA_call = pl.pallas_call(
    _mmA_body,
    grid=(_NB,),
    in_specs=[_deg_spec, _deg1_spec, _row_spec, _w_spec, _b_spec],
    out_specs=[_half_spec, _row_spec],
    out_shape=[jax.ShapeDtypeStruct((2, NP, DH), jnp.float32),
               jax.ShapeDtypeStruct((N, D), jnp.float32)],
)

_mmB_call = pl.pallas_call(
    _mmB_body,
    grid=(_NB,),
    in_specs=[_deg_spec, _deg1_spec, _half_spec, _row_spec, _w_spec, _b_spec],
    out_specs=[_half_spec, _row_spec],
    out_shape=[jax.ShapeDtypeStruct((2, NP, DH), jnp.float32),
               jax.ShapeDtypeStruct((N, D), jnp.float32)],
)

_mmC_call = pl.pallas_call(
    _mmC_body,
    grid=(_NB,),
    in_specs=[_deg_spec, _deg1_spec, _half_spec, _row_spec],
    out_specs=_row_spec,
    out_shape=jax.ShapeDtypeStruct((N, D), jnp.float32),
)


@jax.jit
def kernel(x, edge_index, W1, b1, W2, b2):
    row = edge_index[0].astype(jnp.int32)
    col = edge_index[1].astype(jnp.int32)
    pad = EP - E
    pr = jnp.arange(pad, dtype=jnp.int32)
    row_p = jnp.concatenate([row, pr % N])
    col_p = jnp.concatenate([col, N + pr % (NP - N)])
    col_deg = col_p.reshape(2 * NT, NCH_DEG, CHUNK)
    col_seg = col_p.reshape(NT, NCH, CHUNK)
    row2 = jnp.stack([row_p, row_p + NP]).reshape(2 * NT, NCH, CHUNK)

    deg2 = _deg_call(col_deg)

    y1, a1 = _mmA_call(deg2, deg2, x, W1, b1)
    s1 = _segsum_call(y1.reshape(2 * NP, DH), row2, col_seg)
    y2, a2 = _mmB_call(deg2, deg2, s1.reshape(2, NP, DH), a1, W2, b2)
    s2 = _segsum_call(y2.reshape(2 * NP, DH), row2, col_seg)
    return _mmC_call(deg2, deg2, s2.reshape(2, NP, DH), a2)
